# Initial kernel scaffold; baseline (speedup 1.0000x reference)
import jax
import jax.numpy as jnp
from jax.experimental import pallas as pl


def kernel(features, edge_index, W1, W2, b2):
    raise NotImplementedError("write your pallas kernel here")


# speedup vs baseline: 7.4124x; 7.4124x over previous
import functools

import jax
import jax.numpy as jnp
from jax import lax
from jax.experimental import pallas as pl
from jax.experimental.pallas import tpu as pltpu
from jax.experimental.pallas import tpu_sc as plsc

N_NODES = 10000
N_EDGES = 320000
NFEAT = 128
NCLASS = 40
DPAD = 64

NC = 2
NS = 16
CHUNK = 125
EDGES_PER_TILE = N_EDGES // (NC * NS)
N_CHUNKS = EDGES_PER_TILE // CHUNK
ROWS_PER_TILE = N_NODES // NS


def _make_spmm(D):
  mesh = plsc.VectorSubcoreMesh(core_axis_name="c", subcore_axis_name="s")

  @functools.partial(
      pl.kernel,
      out_type=jax.ShapeDtypeStruct((NC, NS, ROWS_PER_TILE, D), jnp.float32),
      mesh=mesh,
      scratch_types=[
          pltpu.VMEM((N_CHUNKS, CHUNK), jnp.int32),
          pltpu.VMEM((N_CHUNKS, CHUNK), jnp.int32),
          pltpu.VMEM((CHUNK, D), jnp.float32),
          pltpu.VMEM_SHARED((N_NODES, D), jnp.float32),
          pltpu.SemaphoreType.DMA,
      ],
  )
  def spmm(table_hbm, src_hbm, dst_hbm, zeros_hbm, out_hbm, sidx, didx, rows,
           acc, sem):
    c = lax.axis_index("c")
    s = lax.axis_index("s")
    wid = c * NS + s

    pltpu.sync_copy(zeros_hbm.at[s],
                    acc.at[pl.ds(s * ROWS_PER_TILE, ROWS_PER_TILE)])

    pltpu.sync_copy(src_hbm.at[pl.ds(wid * N_CHUNKS, N_CHUNKS)], sidx)
    pltpu.sync_copy(dst_hbm.at[pl.ds(wid * N_CHUNKS, N_CHUNKS)], didx)

    plsc.subcore_barrier()

    def body(i, carry):
      pltpu.async_copy(table_hbm.at[sidx.at[i]], rows, sem).wait()
      pltpu.sync_copy(rows, acc.at[didx.at[i]], add=True)
      return carry

    lax.fori_loop(0, N_CHUNKS, body, 0)

    plsc.subcore_barrier()

    pltpu.sync_copy(acc.at[pl.ds(s * ROWS_PER_TILE, ROWS_PER_TILE)],
                    out_hbm.at[c, s])

  return spmm


_spmm128 = _make_spmm(NFEAT)


_BM = 400


def _mid_body(q0_ref, q1_ref, w1_ref, h_ref):
  x = q0_ref[...] + q1_ref[...]
  h_ref[...] = jnp.maximum(
      jnp.dot(x, w1_ref[...], preferred_element_type=jnp.float32), 0.0)


def _mid(q0, q1, W1):
  return pl.pallas_call(
      _mid_body,
      grid=(N_NODES // _BM,),
      in_specs=[
          pl.BlockSpec((_BM, NFEAT), lambda i: (i, 0)),
          pl.BlockSpec((_BM, NFEAT), lambda i: (i, 0)),
          pl.BlockSpec((NFEAT, NFEAT), lambda i: (0, 0)),
      ],
      out_specs=pl.BlockSpec((_BM, NFEAT), lambda i: (i, 0)),
      out_shape=jax.ShapeDtypeStruct((N_NODES, NFEAT), jnp.float32),
  )(q0, q1, W1)


def _fin_body(r0_ref, r1_ref, w2_ref, b2_ref, o_ref):
  x2 = r0_ref[...] + r1_ref[...]
  y = jnp.dot(x2, w2_ref[...], preferred_element_type=jnp.float32) + b2_ref[...]
  col = lax.broadcasted_iota(jnp.int32, y.shape, 1)
  ym = jnp.where(col < NCLASS, y, -jnp.inf)
  m = jnp.max(ym, axis=1, keepdims=True)
  lse = jnp.log(jnp.sum(jnp.exp(ym - m), axis=1, keepdims=True)) + m
  o_ref[...] = y - lse


def _fin(r0, r1, W2p, b2p):
  return pl.pallas_call(
      _fin_body,
      grid=(N_NODES // _BM,),
      in_specs=[
          pl.BlockSpec((_BM, NFEAT), lambda i: (i, 0)),
          pl.BlockSpec((_BM, NFEAT), lambda i: (i, 0)),
          pl.BlockSpec((NFEAT, DPAD), lambda i: (0, 0)),
          pl.BlockSpec((1, DPAD), lambda i: (0, 0)),
      ],
      out_specs=pl.BlockSpec((_BM, DPAD), lambda i: (i, 0)),
      out_shape=jax.ShapeDtypeStruct((N_NODES, DPAD), jnp.float32),
  )(r0, r1, W2p, b2p)


def kernel(features, edge_index, W1, W2, b2):
  src = edge_index[0].reshape(N_EDGES // CHUNK, CHUNK)
  dst = edge_index[1].reshape(N_EDGES // CHUNK, CHUNK)

  z128 = jnp.zeros((NS, ROWS_PER_TILE, NFEAT), jnp.float32)
  p = _spmm128(features, src, dst, z128).reshape(NC, N_NODES, NFEAT)
  h = _mid(p[0], p[1], W1)
  r = _spmm128(h, src, dst, z128).reshape(NC, N_NODES, NFEAT)
  W2p = jnp.pad(W2, ((0, 0), (0, DPAD - NCLASS)))
  b2p = jnp.pad(b2, (0, DPAD - NCLASS)).reshape(1, DPAD)
  out = _fin(r[0], r[1], W2p, b2p)
  return out[:, :NCLASS]


# speedup vs baseline: 10.2180x; 1.3785x over previous
import functools

import jax
import jax.numpy as jnp
from jax import lax
from jax.experimental import pallas as pl
from jax.experimental.pallas import tpu as pltpu
from jax.experimental.pallas import tpu_sc as plsc

N_NODES = 10000
N_EDGES = 320000
NFEAT = 128
NCLASS = 40
DPAD = 64

NC = 2
NS = 16
CHUNK = 125
EDGES_PER_TILE = N_EDGES // (NC * NS)
N_CHUNKS = EDGES_PER_TILE // CHUNK
HALF = N_CHUNKS // 2
ROWS_PER_TILE = N_NODES // NS


def _make_spmm(D):
  mesh = plsc.VectorSubcoreMesh(core_axis_name="c", subcore_axis_name="s")

  @functools.partial(
      pl.kernel,
      out_type=jax.ShapeDtypeStruct((NC, NS, ROWS_PER_TILE, D), jnp.float32),
      mesh=mesh,
      scratch_types=[
          pltpu.VMEM((HALF, CHUNK), jnp.int32),
          pltpu.VMEM((HALF, CHUNK), jnp.int32),
          pltpu.VMEM((CHUNK, D), jnp.float32),
          pltpu.VMEM((CHUNK, D), jnp.float32),
          pltpu.VMEM_SHARED((N_NODES, D), jnp.float32),
          pltpu.SemaphoreType.DMA,
          pltpu.SemaphoreType.DMA,
      ],
  )
  def spmm(table_hbm, src_hbm, dst_hbm, zeros_hbm, out_hbm, sidx, didx, r0,
           r1, acc, sem0, sem1):
    c = lax.axis_index("c")
    s = lax.axis_index("s")
    wid = c * NS + s

    pltpu.sync_copy(zeros_hbm.at[s],
                    acc.at[pl.ds(s * ROWS_PER_TILE, ROWS_PER_TILE)])

    def wait0():
      pltpu.make_async_copy(table_hbm.at[sidx.at[0]], r0, sem0).wait()

    def wait1():
      pltpu.make_async_copy(table_hbm.at[sidx.at[0]], r1, sem1).wait()

    for h in range(N_CHUNKS // HALF):
      base = wid * N_CHUNKS + h * HALF
      pltpu.sync_copy(src_hbm.at[pl.ds(base, HALF)], sidx)
      pltpu.sync_copy(dst_hbm.at[pl.ds(base, HALF)], didx)
      pltpu.async_copy(table_hbm.at[sidx.at[0]], r0, sem0)
      pltpu.async_copy(table_hbm.at[sidx.at[1]], r1, sem1)
      if h == 0:
        plsc.subcore_barrier()

      def body(i, carry):
        j0 = 2 * i
        wait0()
        pltpu.sync_copy(r0, acc.at[didx.at[j0]], add=True)
        pltpu.async_copy(
            table_hbm.at[sidx.at[jnp.minimum(j0 + 2, HALF - 1)]], r0, sem0)
        wait1()
        pltpu.sync_copy(r1, acc.at[didx.at[j0 + 1]], add=True)
        pltpu.async_copy(
            table_hbm.at[sidx.at[jnp.minimum(j0 + 3, HALF - 1)]], r1, sem1)
        return carry

      lax.fori_loop(0, HALF // 2, body, 0)
      wait0()
      wait1()

    plsc.subcore_barrier()

    pltpu.sync_copy(acc.at[pl.ds(s * ROWS_PER_TILE, ROWS_PER_TILE)],
                    out_hbm.at[c, s])

  return spmm


_spmm128 = _make_spmm(NFEAT)


_BM = 400


def _mid_body(q0_ref, q1_ref, w1_ref, h_ref):
  x = q0_ref[...] + q1_ref[...]
  h_ref[...] = jnp.maximum(
      jnp.dot(x, w1_ref[...], preferred_element_type=jnp.float32), 0.0)


def _mid(q0, q1, W1):
  return pl.pallas_call(
      _mid_body,
      grid=(N_NODES // _BM,),
      in_specs=[
          pl.BlockSpec((_BM, NFEAT), lambda i: (i, 0)),
          pl.BlockSpec((_BM, NFEAT), lambda i: (i, 0)),
          pl.BlockSpec((NFEAT, NFEAT), lambda i: (0, 0)),
      ],
      out_specs=pl.BlockSpec((_BM, NFEAT), lambda i: (i, 0)),
      out_shape=jax.ShapeDtypeStruct((N_NODES, NFEAT), jnp.float32),
  )(q0, q1, W1)


def _fin_body(r0_ref, r1_ref, w2_ref, b2_ref, o_ref):
  x2 = r0_ref[...] + r1_ref[...]
  y = jnp.dot(x2, w2_ref[...], preferred_element_type=jnp.float32) + b2_ref[...]
  col = lax.broadcasted_iota(jnp.int32, y.shape, 1)
  ym = jnp.where(col < NCLASS, y, -jnp.inf)
  m = jnp.max(ym, axis=1, keepdims=True)
  lse = jnp.log(jnp.sum(jnp.exp(ym - m), axis=1, keepdims=True)) + m
  o_ref[...] = y - lse


def _fin(r0, r1, W2p, b2p):
  return pl.pallas_call(
      _fin_body,
      grid=(N_NODES // _BM,),
      in_specs=[
          pl.BlockSpec((_BM, NFEAT), lambda i: (i, 0)),
          pl.BlockSpec((_BM, NFEAT), lambda i: (i, 0)),
          pl.BlockSpec((NFEAT, DPAD), lambda i: (0, 0)),
          pl.BlockSpec((1, DPAD), lambda i: (0, 0)),
      ],
      out_specs=pl.BlockSpec((_BM, DPAD), lambda i: (i, 0)),
      out_shape=jax.ShapeDtypeStruct((N_NODES, DPAD), jnp.float32),
  )(r0, r1, W2p, b2p)


def kernel(features, edge_index, W1, W2, b2):
  src = edge_index[0].reshape(N_EDGES // CHUNK, CHUNK)
  dst = edge_index[1].reshape(N_EDGES // CHUNK, CHUNK)

  z128 = jnp.zeros((NS, ROWS_PER_TILE, NFEAT), jnp.float32)
  p = _spmm128(features, src, dst, z128).reshape(NC, N_NODES, NFEAT)
  h = _mid(p[0], p[1], W1)
  r = _spmm128(h, src, dst, z128).reshape(NC, N_NODES, NFEAT)
  W2p = jnp.pad(W2, ((0, 0), (0, DPAD - NCLASS)))
  b2p = jnp.pad(b2, (0, DPAD - NCLASS)).reshape(1, DPAD)
  out = _fin(r[0], r[1], W2p, b2p)
  return out[:, :NCLASS]


# speedup vs baseline: 10.7693x; 1.0540x over previous
import functools

import jax
import jax.numpy as jnp
from jax import lax
from jax.experimental import pallas as pl
from jax.experimental.pallas import tpu as pltpu
from jax.experimental.pallas import tpu_sc as plsc

N_NODES = 10000
N_EDGES = 320000
NFEAT = 128
NCLASS = 40
DPAD = 64

NC = 2
NS = 16
CHUNK = 125
EDGES_PER_TILE = N_EDGES // (NC * NS)
N_CHUNKS = EDGES_PER_TILE // CHUNK
HALF = N_CHUNKS // 2
ROWS_PER_TILE = N_NODES // NS


def _make_spmm(D):
  mesh = plsc.VectorSubcoreMesh(core_axis_name="c", subcore_axis_name="s")

  @functools.partial(
      pl.kernel,
      out_type=jax.ShapeDtypeStruct((NC, NS, ROWS_PER_TILE, D), jnp.float32),
      mesh=mesh,
      scratch_types=[
          pltpu.VMEM((HALF, CHUNK), jnp.int32),
          pltpu.VMEM((HALF, CHUNK), jnp.int32),
          pltpu.VMEM((CHUNK, D), jnp.float32),
          pltpu.VMEM((CHUNK, D), jnp.float32),
          pltpu.VMEM_SHARED((N_NODES, D), jnp.float32),
          pltpu.SemaphoreType.DMA,
          pltpu.SemaphoreType.DMA,
      ],
  )
  def spmm(table_hbm, src_hbm, dst_hbm, zeros_hbm, out_hbm, sidx, didx, r0,
           r1, acc, sem0, sem1):
    c = lax.axis_index("c")
    s = lax.axis_index("s")
    wid = c * NS + s

    pltpu.sync_copy(zeros_hbm.at[s],
                    acc.at[pl.ds(s * ROWS_PER_TILE, ROWS_PER_TILE)])

    def wait0():
      pltpu.make_async_copy(table_hbm.at[sidx.at[0]], r0, sem0).wait()

    def wait1():
      pltpu.make_async_copy(table_hbm.at[sidx.at[0]], r1, sem1).wait()

    for h in range(N_CHUNKS // HALF):
      base = wid * N_CHUNKS + h * HALF
      pltpu.sync_copy(src_hbm.at[pl.ds(base, HALF)], sidx)
      pltpu.sync_copy(dst_hbm.at[pl.ds(base, HALF)], didx)
      pltpu.async_copy(table_hbm.at[sidx.at[0]], r0, sem0)
      pltpu.async_copy(table_hbm.at[sidx.at[1]], r1, sem1)
      if h == 0:
        plsc.subcore_barrier()

      def body(i, carry):
        j0 = 2 * i
        wait0()
        pltpu.sync_copy(r0, acc.at[didx.at[j0]], add=True)
        pltpu.async_copy(
            table_hbm.at[sidx.at[jnp.minimum(j0 + 2, HALF - 1)]], r0, sem0)
        wait1()
        pltpu.sync_copy(r1, acc.at[didx.at[j0 + 1]], add=True)
        pltpu.async_copy(
            table_hbm.at[sidx.at[jnp.minimum(j0 + 3, HALF - 1)]], r1, sem1)
        return carry

      lax.fori_loop(0, HALF // 2, body, 0)
      wait0()
      wait1()

    plsc.subcore_barrier()

    pltpu.sync_copy(acc.at[pl.ds(s * ROWS_PER_TILE, ROWS_PER_TILE)],
                    out_hbm.at[c, s])

  return spmm


_spmm128 = _make_spmm(NFEAT)


_BM = 400


def _mid_body(q0_ref, q1_ref, w1_ref, h_ref):
  x = q0_ref[...] + q1_ref[...]
  h_ref[...] = jnp.maximum(
      jnp.dot(x, w1_ref[...], preferred_element_type=jnp.float32), 0.0)


def _mid(p, W1):
  nb = N_NODES // _BM
  return pl.pallas_call(
      _mid_body,
      grid=(nb,),
      in_specs=[
          pl.BlockSpec((_BM, NFEAT), lambda i: (i, 0)),
          pl.BlockSpec((_BM, NFEAT), lambda i, _nb=nb: (i + _nb, 0)),
          pl.BlockSpec((NFEAT, NFEAT), lambda i: (0, 0)),
      ],
      out_specs=pl.BlockSpec((_BM, NFEAT), lambda i: (i, 0)),
      out_shape=jax.ShapeDtypeStruct((N_NODES, NFEAT), jnp.float32),
  )(p, p, W1)


def _fin_body(r0_ref, r1_ref, w2_ref, b2_ref, o_ref):
  x2 = r0_ref[...] + r1_ref[...]
  y = jnp.dot(x2, w2_ref[...], preferred_element_type=jnp.float32) + b2_ref[...]
  col = lax.broadcasted_iota(jnp.int32, y.shape, 1)
  ym = jnp.where(col < NCLASS, y, -jnp.inf)
  m = jnp.max(ym, axis=1, keepdims=True)
  lse = jnp.log(jnp.sum(jnp.exp(ym - m), axis=1, keepdims=True)) + m
  o_ref[...] = (y - lse)[:, :NCLASS]


def _fin(r, W2p, b2p):
  nb = N_NODES // _BM
  return pl.pallas_call(
      _fin_body,
      grid=(nb,),
      in_specs=[
          pl.BlockSpec((_BM, NFEAT), lambda i: (i, 0)),
          pl.BlockSpec((_BM, NFEAT), lambda i, _nb=nb: (i + _nb, 0)),
          pl.BlockSpec((NFEAT, DPAD), lambda i: (0, 0)),
          pl.BlockSpec((1, DPAD), lambda i: (0, 0)),
      ],
      out_specs=pl.BlockSpec((_BM, NCLASS), lambda i: (i, 0)),
      out_shape=jax.ShapeDtypeStruct((N_NODES, NCLASS), jnp.float32),
  )(r, r, W2p, b2p)


def kernel(features, edge_index, W1, W2, b2):
  src = edge_index[0].reshape(N_EDGES // CHUNK, CHUNK)
  dst = edge_index[1].reshape(N_EDGES // CHUNK, CHUNK)

  z128 = jnp.zeros((NS, ROWS_PER_TILE, NFEAT), jnp.float32)
  p = _spmm128(features, src, dst, z128).reshape(NC * N_NODES, NFEAT)
  h = _mid(p, W1)
  r = _spmm128(h, src, dst, z128).reshape(NC * N_NODES, NFEAT)
  W2p = jnp.pad(W2, ((0, 0), (0, DPAD - NCLASS)))
  b2p = jnp.pad(b2, (0, DPAD - NCLASS)).reshape(1, DPAD)
  return _fin(r, W2p, b2p)


# speedup vs baseline: 12.0456x; 1.1185x over previous
import functools

import jax
import jax.numpy as jnp
from jax import lax
from jax.experimental import pallas as pl
from jax.experimental.pallas import tpu as pltpu
from jax.experimental.pallas import tpu_sc as plsc

N_NODES = 10000
N_EDGES = 320000
NFEAT = 128
NCLASS = 40
DPAD = 64

NC = 2
NS = 16
CHUNK = 125
EDGES_PER_TILE = N_EDGES // (NC * NS)
N_CHUNKS = EDGES_PER_TILE // CHUNK
HALF = N_CHUNKS // 2
ROWS_PER_TILE = N_NODES // NS


def _make_spmm(D, tc_tiling=True):
  mesh = plsc.VectorSubcoreMesh(core_axis_name="c", subcore_axis_name="s")
  params = None if tc_tiling else pltpu.CompilerParams(use_tc_tiling_on_sc=False)

  @functools.partial(
      pl.kernel,
      out_type=jax.ShapeDtypeStruct((NC, NS, ROWS_PER_TILE, D), jnp.float32),
      compiler_params=params,
      mesh=mesh,
      scratch_types=[
          pltpu.VMEM((HALF, CHUNK), jnp.int32),
          pltpu.VMEM((HALF, CHUNK), jnp.int32),
          pltpu.VMEM((CHUNK, D), jnp.float32),
          pltpu.VMEM((CHUNK, D), jnp.float32),
          pltpu.VMEM_SHARED((N_NODES, D), jnp.float32),
          pltpu.SemaphoreType.DMA,
          pltpu.SemaphoreType.DMA,
      ],
  )
  def spmm(table_hbm, src_hbm, dst_hbm, zeros_hbm, out_hbm, sidx, didx, r0,
           r1, acc, sem0, sem1):
    c = lax.axis_index("c")
    s = lax.axis_index("s")
    wid = c * NS + s

    pltpu.sync_copy(zeros_hbm.at[s],
                    acc.at[pl.ds(s * ROWS_PER_TILE, ROWS_PER_TILE)])

    def wait0():
      pltpu.make_async_copy(table_hbm.at[sidx.at[0]], r0, sem0).wait()

    def wait1():
      pltpu.make_async_copy(table_hbm.at[sidx.at[0]], r1, sem1).wait()

    for h in range(N_CHUNKS // HALF):
      base = wid * N_CHUNKS + h * HALF
      pltpu.sync_copy(src_hbm.at[pl.ds(base, HALF)], sidx)
      pltpu.sync_copy(dst_hbm.at[pl.ds(base, HALF)], didx)
      pltpu.async_copy(table_hbm.at[sidx.at[0]], r0, sem0)
      pltpu.async_copy(table_hbm.at[sidx.at[1]], r1, sem1)
      if h == 0:
        plsc.subcore_barrier()

      def body(i, carry):
        j0 = 2 * i
        wait0()
        pltpu.sync_copy(r0, acc.at[didx.at[j0]], add=True)
        pltpu.async_copy(
            table_hbm.at[sidx.at[jnp.minimum(j0 + 2, HALF - 1)]], r0, sem0)
        wait1()
        pltpu.sync_copy(r1, acc.at[didx.at[j0 + 1]], add=True)
        pltpu.async_copy(
            table_hbm.at[sidx.at[jnp.minimum(j0 + 3, HALF - 1)]], r1, sem1)
        return carry

      lax.fori_loop(0, HALF // 2, body, 0)
      wait0()
      wait1()

    plsc.subcore_barrier()

    pltpu.sync_copy(acc.at[pl.ds(s * ROWS_PER_TILE, ROWS_PER_TILE)],
                    out_hbm.at[c, s])

  return spmm


_spmm128 = _make_spmm(NFEAT)
_spmm64 = _make_spmm(DPAD, tc_tiling=False)


_BM = 400


def _mid_body(q0_ref, q1_ref, w1_ref, w2_ref, g_ref):
  x = q0_ref[...] + q1_ref[...]
  h = jnp.maximum(
      jnp.dot(x, w1_ref[...], preferred_element_type=jnp.float32), 0.0)
  g_ref[...] = jnp.dot(h, w2_ref[...], preferred_element_type=jnp.float32)


def _mid(p, W1, W2p):
  nb = N_NODES // _BM
  return pl.pallas_call(
      _mid_body,
      grid=(nb,),
      in_specs=[
          pl.BlockSpec((_BM, NFEAT), lambda i: (i, 0)),
          pl.BlockSpec((_BM, NFEAT), lambda i, _nb=nb: (i + _nb, 0)),
          pl.BlockSpec((NFEAT, NFEAT), lambda i: (0, 0)),
          pl.BlockSpec((NFEAT, DPAD), lambda i: (0, 0)),
      ],
      out_specs=pl.BlockSpec((_BM, DPAD), lambda i: (i, 0)),
      out_shape=jax.ShapeDtypeStruct((N_NODES, DPAD), jnp.float32),
  )(p, p, W1, W2p)


def _fin_body(r0_ref, r1_ref, b2_ref, o_ref):
  y = r0_ref[...] + r1_ref[...] + b2_ref[...]
  col = lax.broadcasted_iota(jnp.int32, y.shape, 1)
  ym = jnp.where(col < NCLASS, y, -jnp.inf)
  m = jnp.max(ym, axis=1, keepdims=True)
  lse = jnp.log(jnp.sum(jnp.exp(ym - m), axis=1, keepdims=True)) + m
  o_ref[...] = (y - lse)[:, :NCLASS]


def _fin(r, b2p):
  nb = N_NODES // _BM
  return pl.pallas_call(
      _fin_body,
      grid=(nb,),
      in_specs=[
          pl.BlockSpec((_BM, DPAD), lambda i: (i, 0)),
          pl.BlockSpec((_BM, DPAD), lambda i, _nb=nb: (i + _nb, 0)),
          pl.BlockSpec((1, DPAD), lambda i: (0, 0)),
      ],
      out_specs=pl.BlockSpec((_BM, NCLASS), lambda i: (i, 0)),
      out_shape=jax.ShapeDtypeStruct((N_NODES, NCLASS), jnp.float32),
  )(r, r, b2p)


def kernel(features, edge_index, W1, W2, b2):
  src = edge_index[0].reshape(N_EDGES // CHUNK, CHUNK)
  dst = edge_index[1].reshape(N_EDGES // CHUNK, CHUNK)

  z128 = jnp.zeros((NS, ROWS_PER_TILE, NFEAT), jnp.float32)
  z64 = jnp.zeros((NS, ROWS_PER_TILE, DPAD), jnp.float32)
  p = _spmm128(features, src, dst, z128).reshape(NC * N_NODES, NFEAT)
  W2p = jnp.pad(W2, ((0, 0), (0, DPAD - NCLASS)))
  g = _mid(p, W1, W2p)
  r = _spmm64(g, src, dst, z64).reshape(NC * N_NODES, DPAD)
  b2p = jnp.pad(b2, (0, DPAD - NCLASS)).reshape(1, DPAD)
  return _fin(r, b2p)


# speedup vs baseline: 14.0951x; 1.1701x over previous
import functools

import jax
import jax.numpy as jnp
from jax import lax
from jax.experimental import pallas as pl
from jax.experimental.pallas import tpu as pltpu
from jax.experimental.pallas import tpu_sc as plsc

N_NODES = 10000
N_EDGES = 320000
NFEAT = 128
NCLASS = 40
DPAD = 64

NC = 2
NS = 16
CHUNK = 125
EDGES_PER_TILE = N_EDGES // (NC * NS)
N_CHUNKS = EDGES_PER_TILE // CHUNK
HALF = N_CHUNKS // 2
ROWS_PER_TILE = N_NODES // NS
ZCOPIES = ROWS_PER_TILE // CHUNK


def _make_spmm(D):
  mesh = plsc.VectorSubcoreMesh(core_axis_name="c", subcore_axis_name="s")
  params = pltpu.CompilerParams(use_tc_tiling_on_sc=False)

  @functools.partial(
      pl.kernel,
      out_type=jax.ShapeDtypeStruct((NC * N_NODES, D), jnp.float32),
      compiler_params=params,
      mesh=mesh,
      scratch_types=[
          pltpu.VMEM((HALF, CHUNK), jnp.int32),
          pltpu.VMEM((HALF, CHUNK), jnp.int32),
          pltpu.VMEM((CHUNK, D), jnp.float32),
          pltpu.VMEM((CHUNK, D), jnp.float32),
          pltpu.VMEM_SHARED((N_NODES, D), jnp.float32),
          pltpu.SemaphoreType.DMA,
          pltpu.SemaphoreType.DMA,
      ],
  )
  def spmm(table_hbm, ei_hbm, out_hbm, sidx, didx, r0, r1, acc, sem0, sem1):
    c = lax.axis_index("c")
    s = lax.axis_index("s")
    wid = c * NS + s

    zero16 = jnp.zeros((16,), jnp.float32)

    def zrow(i, carry):
      for j in range(D // 16):
        r0[i, pl.ds(j * 16, 16)] = zero16
      return carry

    lax.fori_loop(0, CHUNK, zrow, 0)
    for k in range(ZCOPIES):
      pltpu.sync_copy(r0, acc.at[pl.ds((s * ZCOPIES + k) * CHUNK, CHUNK)])

    def wait0():
      pltpu.make_async_copy(table_hbm.at[sidx.at[0]], r0, sem0).wait()

    def wait1():
      pltpu.make_async_copy(table_hbm.at[sidx.at[0]], r1, sem1).wait()

    for h in range(N_CHUNKS // HALF):
      base = wid * N_CHUNKS + h * HALF
      pltpu.sync_copy(ei_hbm.at[0, pl.ds(base, HALF)], sidx)
      pltpu.sync_copy(ei_hbm.at[1, pl.ds(base, HALF)], didx)
      pltpu.async_copy(table_hbm.at[sidx.at[0]], r0, sem0)
      pltpu.async_copy(table_hbm.at[sidx.at[1]], r1, sem1)
      if h == 0:
        plsc.subcore_barrier()

      def body(i, carry):
        j0 = 2 * i
        wait0()
        pltpu.sync_copy(r0, acc.at[didx.at[j0]], add=True)
        pltpu.async_copy(
            table_hbm.at[sidx.at[jnp.minimum(j0 + 2, HALF - 1)]], r0, sem0)
        wait1()
        pltpu.sync_copy(r1, acc.at[didx.at[j0 + 1]], add=True)
        pltpu.async_copy(
            table_hbm.at[sidx.at[jnp.minimum(j0 + 3, HALF - 1)]], r1, sem1)
        return carry

      lax.fori_loop(0, HALF // 2, body, 0)
      wait0()
      wait1()

    plsc.subcore_barrier()

    pltpu.sync_copy(
        acc.at[pl.ds(s * ROWS_PER_TILE, ROWS_PER_TILE)],
        out_hbm.at[pl.ds(c * N_NODES + s * ROWS_PER_TILE, ROWS_PER_TILE)])

  return spmm


_spmm128 = _make_spmm(NFEAT)
_spmm64 = _make_spmm(DPAD)


_BM = 1000
_NB = N_NODES // _BM


def _mid_body(q0_ref, q1_ref, w1_ref, w2_ref, g_ref):
  x = q0_ref[...] + q1_ref[...]
  h = jnp.maximum(
      jnp.dot(x, w1_ref[...], preferred_element_type=jnp.float32), 0.0)
  g_ref[...] = jnp.dot(h, w2_ref[...], preferred_element_type=jnp.float32)


def _mid(p, W1, W2p):
  return pl.pallas_call(
      _mid_body,
      grid=(_NB,),
      in_specs=[
          pl.BlockSpec((_BM, NFEAT), lambda i: (i, 0)),
          pl.BlockSpec((_BM, NFEAT), lambda i: (i + _NB, 0)),
          pl.BlockSpec((NFEAT, NFEAT), lambda i: (0, 0)),
          pl.BlockSpec((NFEAT, DPAD), lambda i: (0, 0)),
      ],
      out_specs=pl.BlockSpec((_BM, DPAD), lambda i: (i, 0)),
      out_shape=jax.ShapeDtypeStruct((N_NODES, DPAD), jnp.float32),
  )(p, p, W1, W2p)


def _fin_body(r0_ref, r1_ref, b2_ref, o_ref):
  y = r0_ref[...] + r1_ref[...] + b2_ref[...]
  col = lax.broadcasted_iota(jnp.int32, y.shape, 1)
  ym = jnp.where(col < NCLASS, y, -jnp.inf)
  m = jnp.max(ym, axis=1, keepdims=True)
  lse = jnp.log(jnp.sum(jnp.exp(ym - m), axis=1, keepdims=True)) + m
  o_ref[...] = (y - lse)[:, :NCLASS]


def _fin(r, b2p):
  return pl.pallas_call(
      _fin_body,
      grid=(_NB,),
      in_specs=[
          pl.BlockSpec((_BM, DPAD), lambda i: (i, 0)),
          pl.BlockSpec((_BM, DPAD), lambda i: (i + _NB, 0)),
          pl.BlockSpec((1, DPAD), lambda i: (0, 0)),
      ],
      out_specs=pl.BlockSpec((_BM, NCLASS), lambda i: (i, 0)),
      out_shape=jax.ShapeDtypeStruct((N_NODES, NCLASS), jnp.float32),
  )(r, r, b2p)


def kernel(features, edge_index, W1, W2, b2):
  ei = edge_index.reshape(2, N_EDGES // CHUNK, CHUNK)
  p = _spmm128(features, ei)
  W2p = jnp.pad(W2, ((0, 0), (0, DPAD - NCLASS)))
  g = _mid(p, W1, W2p)
  r = _spmm64(g, ei)
  b2p = jnp.pad(b2, (0, DPAD - NCLASS)).reshape(1, DPAD)
  return _fin(r, b2p)
